# Initial kernel scaffold; baseline (speedup 1.0000x reference)
import functools

import jax
import jax.numpy as jnp
from jax import lax
from jax.experimental import pallas as pl
from jax.experimental.pallas import tpu as pltpu
from jax.experimental.pallas import tpu_sc as plsc

N = 50000
NA = 50016
E = 800000
NC, NS = 2, 16
NW = NC * NS
B = 128
KB = 196
EPAD = NW * KB * B
F = 32
RB = 1000
GRID = N // RB
ZR = 625
EPS = 1e-5

_MESH = plsc.VectorSubcoreMesh(
    core_axis_name="c", subcore_axis_name="s", num_cores=NC, num_subcores=NS)


def _deg_body(dst_hbm, ones_hbm, z_hbm, out_hbm, dst_v, ones_v, zeros_v, acc):
    cc = lax.axis_index("c")
    sid = lax.axis_index("s")
    wid = cc * NS + sid
    pltpu.sync_copy(dst_hbm.at[wid], dst_v)
    pltpu.sync_copy(ones_hbm, ones_v)
    pltpu.sync_copy(z_hbm, zeros_v)
    for r in range(5):
        pltpu.sync_copy(zeros_v, acc.at[pl.ds(sid * 3125 + r * ZR, ZR)])
    plsc.subcore_barrier()

    def body(j, car):
        pltpu.sync_copy(ones_v, acc.at[dst_v.at[j]], add=True)
        return car

    lax.fori_loop(0, KB, body, 0)
    plsc.subcore_barrier()
    pltpu.sync_copy(acc.at[pl.ds(sid * 3125, 3125)],
                    out_hbm.at[cc, pl.ds(sid * 3125, 3125)])


_deg_kernel = functools.partial(
    pl.kernel,
    out_type=jax.ShapeDtypeStruct((NC, N, 16), jnp.float32),
    mesh=_MESH,
    scratch_types=[
        pltpu.VMEM((KB, B), jnp.int32),
        pltpu.VMEM((B, 16), jnp.float32),
        pltpu.VMEM((ZR, 16), jnp.float32),
        pltpu.VMEM_SHARED((NA, 16), jnp.float32),
    ],
)(_deg_body)


def _make_scatter(C):
    def body(*refs):
        src_hbm, dst_hbm, z_hbm = refs[0], refs[1], refs[2]
        tabs = refs[3:3 + C]
        outs = refs[3 + C:3 + 2 * C]
        src_v, dst_v, zeros_v, buf, acc = refs[3 + 2 * C:]
        cc = lax.axis_index("c")
        sid = lax.axis_index("s")
        wid = cc * NS + sid
        pltpu.sync_copy(src_hbm.at[wid], src_v)
        pltpu.sync_copy(dst_hbm.at[wid], dst_v)
        pltpu.sync_copy(z_hbm, zeros_v)
        for c in range(C):
            for r in range(5):
                pltpu.sync_copy(zeros_v, acc.at[pl.ds(sid * 3125 + r * ZR, ZR)])
            plsc.subcore_barrier()
            tab = tabs[c]

            def bat(j, car):
                pltpu.sync_copy(tab.at[src_v.at[j]], buf)
                pltpu.sync_copy(buf, acc.at[dst_v.at[j]], add=True)
                return car

            lax.fori_loop(0, KB, bat, 0)
            plsc.subcore_barrier()
            pltpu.sync_copy(acc.at[pl.ds(sid * 3125, 3125)],
                            outs[c].at[cc, pl.ds(sid * 3125, 3125)])
            plsc.subcore_barrier()

    return pl.kernel(
        body,
        out_type=[jax.ShapeDtypeStruct((NC, N, F), jnp.float32)] * C,
        mesh=_MESH,
        scratch_types=[
            pltpu.VMEM((KB, B), jnp.int32),
            pltpu.VMEM((KB, B), jnp.int32),
            pltpu.VMEM((ZR, F), jnp.float32),
            pltpu.VMEM((B, F), jnp.float32),
            pltpu.VMEM_SHARED((NA, F), jnp.float32),
        ],
    )


_scatter = {C: _make_scatter(C) for C in (1, 2, 4)}


def _row_spec(shape):
    nd = len(shape)
    if nd == 2:
        return pl.BlockSpec((RB, shape[1]), lambda i: (i, 0))
    return pl.BlockSpec((shape[0], RB, shape[2]), lambda i: (0, i, 0))


def _full_spec(shape):
    return pl.BlockSpec(shape, lambda i: (0,) * len(shape))


def _k0_body(x_ref, dA_ref, dB_ref, W_ref, dinv_ref, t0_ref, t1_ref):
    deg = dA_ref[...][:, 0:1] + dB_ref[...][:, 0:1] + 1.0
    dinv = lax.rsqrt(deg)
    dinv_ref[...] = dinv
    xw = jnp.dot(x_ref[...], W_ref[...],
                 preferred_element_type=jnp.float32) * dinv
    t0_ref[...] = xw[:, 0:F]
    t1_ref[...] = xw[:, F:2 * F]


def _k0(x, degp, W1):
    return pl.pallas_call(
        _k0_body,
        grid=(GRID,),
        in_specs=[_row_spec(x.shape),
                  pl.BlockSpec((RB, 16), lambda i: (i, 0)),
                  pl.BlockSpec((RB, 16), lambda i: (i, 0)),
                  _full_spec(W1.shape)],
        out_specs=[_row_spec((N, 1))] + [_row_spec((NA, F))] * 2,
        out_shape=[jax.ShapeDtypeStruct((N, 1), jnp.float32)]
        + [jax.ShapeDtypeStruct((NA, F), jnp.float32)] * 2,
    )(x, degp[0], degp[1], W1)


def _pre_act(dinv_ref, b_ref, p_refs, t_refs):
    parts = [p[...][0] + p[...][1] + t[...] for p, t in zip(p_refs, t_refs)]
    t = parts[0] if len(parts) == 1 else jnp.concatenate(parts, axis=1)
    return t * dinv_ref[...] + b_ref[...]


def _make_stats(C):
    do = F * C

    def body(*refs):
        dinv_ref, b_ref = refs[0], refs[1]
        p_refs = refs[2:2 + C]
        t_refs = refs[2 + C:2 + 2 * C]
        stats_ref, acc_ref = refs[2 + 2 * C], refs[3 + 2 * C]
        i = pl.program_id(0)
        pre = _pre_act(dinv_ref, b_ref, p_refs, t_refs)

        @pl.when(i == 0)
        def _():
            acc_ref[...] = jnp.zeros_like(acc_ref)

        acc_ref[...] += jnp.stack(
            [jnp.sum(pre, axis=0), jnp.sum(pre * pre, axis=0)])

        @pl.when(i == GRID - 1)
        def _():
            stats_ref[...] = acc_ref[...]

    def call(dinv, b, parts, tabs):
        return pl.pallas_call(
            body,
            grid=(GRID,),
            in_specs=[_row_spec((N, 1)), _full_spec((1, do))]
            + [_row_spec((NC, N, F))] * C + [_row_spec((NA, F))] * C,
            out_specs=_full_spec((2, do)),
            out_shape=jax.ShapeDtypeStruct((2, do), jnp.float32),
            scratch_shapes=[pltpu.VMEM((2, do), jnp.float32)],
        )(dinv, b, *parts, *tabs)

    return call


def _bn_apply(pre, stats_ref, g_ref, be_ref):
    m = stats_ref[...][0:1, :] / N
    v = stats_ref[...][1:2, :] / N - m * m
    rstd = lax.rsqrt(v + EPS)
    return (pre - m) * rstd * g_ref[...] + be_ref[...]


def _make_apply(C, C_next):
    do = F * C

    def body(*refs):
        dinv_ref, b_ref, g_ref, be_ref, stats_ref, W_ref = refs[:6]
        p_refs = refs[6:6 + C]
        t_refs = refs[6 + C:6 + 2 * C]
        o_refs = refs[6 + 2 * C:]
        pre = _pre_act(dinv_ref, b_ref, p_refs, t_refs)
        h = jnp.maximum(_bn_apply(pre, stats_ref, g_ref, be_ref), 0.0)
        xw = jnp.dot(h, W_ref[...],
                     preferred_element_type=jnp.float32) * dinv_ref[...]
        for c2 in range(C_next):
            o_refs[c2][...] = xw[:, F * c2:F * (c2 + 1)]

    def call(dinv, b, g, be, stats, W, parts, tabs):
        return pl.pallas_call(
            body,
            grid=(GRID,),
            in_specs=[_row_spec((N, 1)), _full_spec((1, do)),
                      _full_spec((1, do)), _full_spec((1, do)),
                      _full_spec((2, do)), _full_spec(W.shape)]
            + [_row_spec((NC, N, F))] * C + [_row_spec((NA, F))] * C,
            out_specs=[_row_spec((NA, F))] * C_next,
            out_shape=[jax.ShapeDtypeStruct((NA, F), jnp.float32)] * C_next,
        )(dinv, b, g, be, stats, W, *parts, *tabs)

    return call


def _apply4_body(dinv_ref, b_ref, g_ref, be_ref, stats_ref, cW_ref, cb_ref,
                 p_ref, t_ref, d1_ref, cstats_ref, acc_ref):
    i = pl.program_id(0)
    pre = _pre_act(dinv_ref, b_ref, [p_ref], [t_ref])
    h4 = jnp.maximum(_bn_apply(pre, stats_ref, g_ref, be_ref), 0.0)
    d1 = jnp.maximum(
        jnp.dot(h4, cW_ref[...], preferred_element_type=jnp.float32)
        + cb_ref[...], 0.0)
    d1_ref[...] = d1

    @pl.when(i == 0)
    def _():
        acc_ref[...] = jnp.zeros_like(acc_ref)

    acc_ref[...] += jnp.stack([jnp.sum(d1, axis=0), jnp.sum(d1 * d1, axis=0)])

    @pl.when(i == GRID - 1)
    def _():
        cstats_ref[...] = acc_ref[...]


def _apply4(dinv, b, g, be, stats, cW1, cb1, part, tab):
    return pl.pallas_call(
        _apply4_body,
        grid=(GRID,),
        in_specs=[_row_spec((N, 1)), _full_spec((1, F)), _full_spec((1, F)),
                  _full_spec((1, F)), _full_spec((2, F)),
                  _full_spec(cW1.shape), _full_spec((1, F)),
                  _row_spec((NC, N, F)), _row_spec((NA, F))],
        out_specs=[_row_spec((N, F)), _full_spec((2, F))],
        out_shape=[jax.ShapeDtypeStruct((N, F), jnp.float32),
                   jax.ShapeDtypeStruct((2, F), jnp.float32)],
        scratch_shapes=[pltpu.VMEM((2, F), jnp.float32)],
    )(dinv, b, g, be, stats, cW1, cb1, part, tab)


def _mlp_body(d_ref, stats_ref, g_ref, be_ref, W_ref, wb_ref,
              d2_ref, cstats_ref, acc_ref):
    i = pl.program_id(0)
    e = _bn_apply(d_ref[...], stats_ref, g_ref, be_ref)
    d2 = jnp.maximum(
        jnp.dot(e, W_ref[...], preferred_element_type=jnp.float32)
        + wb_ref[...], 0.0)
    d2_ref[...] = d2

    @pl.when(i == 0)
    def _():
        acc_ref[...] = jnp.zeros_like(acc_ref)

    acc_ref[...] += jnp.stack([jnp.sum(d2, axis=0), jnp.sum(d2 * d2, axis=0)])

    @pl.when(i == GRID - 1)
    def _():
        cstats_ref[...] = acc_ref[...]


def _mlp(d, stats, g, be, W, wb, dn):
    di = d.shape[1]
    return pl.pallas_call(
        _mlp_body,
        grid=(GRID,),
        in_specs=[_row_spec((N, di)), _full_spec((2, di)),
                  _full_spec((1, di)), _full_spec((1, di)),
                  _full_spec(W.shape), _full_spec((1, dn))],
        out_specs=[_row_spec((N, dn)), _full_spec((2, dn))],
        out_shape=[jax.ShapeDtypeStruct((N, dn), jnp.float32),
                   jax.ShapeDtypeStruct((2, dn), jnp.float32)],
        scratch_shapes=[pltpu.VMEM((2, dn), jnp.float32)],
    )(d, stats, g, be, W, wb)


def _final_body(d_ref, stats_ref, g_ref, be_ref, W_ref, wb_ref, out_ref):
    e = _bn_apply(d_ref[...], stats_ref, g_ref, be_ref)
    out_ref[...] = (jnp.dot(e, W_ref[...], preferred_element_type=jnp.float32)
                    + wb_ref[...])


def _final(d, stats, g, be, W, wb):
    di, dn = W.shape
    return pl.pallas_call(
        _final_body,
        grid=(GRID,),
        in_specs=[_row_spec((N, di)), _full_spec((2, di)),
                  _full_spec((1, di)), _full_spec((1, di)),
                  _full_spec(W.shape), _full_spec((1, dn))],
        out_specs=_row_spec((N, dn)),
        out_shape=jax.ShapeDtypeStruct((N, dn), jnp.float32),
    )(d, stats, g, be, W, wb)


def kernel(x, edge_index, W1, b1, g1, be1, W2, b2, g2, be2, W3, b3, g3, be3,
           W4, b4, g4, be4, cW1, cb1, cW2, cb2, cW3, cb3, cg1, cbe1,
           cg2, cbe2):
    pad = jnp.full((EPAD - E,), N, jnp.int32)
    src_t = jnp.concatenate([edge_index[0], pad]).reshape(NW, KB, B)
    dst_t = jnp.concatenate([edge_index[1], pad]).reshape(NW, KB, B)
    ones16 = jnp.ones((B, 16), jnp.float32)
    z16 = jnp.zeros((ZR, 16), jnp.float32)
    zF = jnp.zeros((ZR, F), jnp.float32)
    r2 = lambda v: v.reshape(1, -1)

    degp = _deg_kernel(dst_t, ones16, z16)
    dinv, ta, tb = _k0(x, degp, W1)
    tabs = [ta, tb]

    layer_params = [(b1, g1, be1, 2, W2, 4), (b2, g2, be2, 4, W3, 2),
                    (b3, g3, be3, 2, W4, 1)]
    for b_, g_, be_, C, Wn, Cn in layer_params:
        parts = _scatter[C](src_t, dst_t, zF, *tabs)
        parts = list(parts) if isinstance(parts, (list, tuple)) else [parts]
        stats = _make_stats(C)(dinv, r2(b_), parts, tabs)
        tabs = _make_apply(C, Cn)(dinv, r2(b_), r2(g_), r2(be_), stats, Wn,
                                  parts, tabs)
        tabs = list(tabs) if isinstance(tabs, (list, tuple)) else [tabs]

    parts = _scatter[1](src_t, dst_t, zF, *tabs)
    parts = list(parts) if isinstance(parts, (list, tuple)) else [parts]
    stats4 = _make_stats(1)(dinv, r2(b4), parts, tabs)
    d1, cs1 = _apply4(dinv, r2(b4), r2(g4), r2(be4), stats4, cW1, r2(cb1),
                      parts[0], tabs[0])
    d2, cs2 = _mlp(d1, cs1, r2(cg1), r2(cbe1), cW2, r2(cb2), 16)
    out = _final(d2, cs2, r2(cg2), r2(cbe2), cW3, r2(cb3))
    return out


# speedup vs baseline: 5.1628x; 5.1628x over previous
import functools

import jax
import jax.numpy as jnp
from jax import lax
from jax.experimental import pallas as pl
from jax.experimental.pallas import tpu as pltpu
from jax.experimental.pallas import tpu_sc as plsc

N = 50000
NA = 50048
E = 800000
NC, NS = 2, 16
NW = NC * NS
B = 128
KB = 200
EPAD = NW * KB * B
F = 16
RB = 1000
GRID = N // RB
EPS = 1e-5

_MESH = plsc.VectorSubcoreMesh(
    core_axis_name="c", subcore_axis_name="s", num_cores=NC, num_subcores=NS)


def _zero_acc(zeros_v, acc, sid):
    base = sid * 3128
    for r in range(24):
        pltpu.sync_copy(zeros_v, acc.at[pl.ds(base + r * 128, 128)])
    pltpu.sync_copy(zeros_v.at[pl.ds(0, 56)], acc.at[pl.ds(base + 3072, 56)])


def _deg_body(dst_hbm, ones_hbm, z_hbm, out_hbm, dst_v, ones_v, zeros_v, acc):
    cc = lax.axis_index("c")
    sid = lax.axis_index("s")
    wid = cc * NS + sid
    pltpu.sync_copy(dst_hbm.at[wid], dst_v)
    pltpu.sync_copy(ones_hbm, ones_v)
    pltpu.sync_copy(z_hbm, zeros_v)
    _zero_acc(zeros_v, acc, sid)
    plsc.subcore_barrier()

    def body(j, car):
        pltpu.sync_copy(ones_v, acc.at[dst_v.at[j]], add=True)
        return car

    lax.fori_loop(0, KB, body, 0)
    plsc.subcore_barrier()
    pltpu.sync_copy(acc.at[pl.ds(sid * 3128, 3128)],
                    out_hbm.at[cc, pl.ds(sid * 3128, 3128)])


_SC_PARAMS = pltpu.CompilerParams(use_tc_tiling_on_sc=False)

_deg_kernel = functools.partial(
    pl.kernel,
    out_type=jax.ShapeDtypeStruct((NC, NA, 16), jnp.float32),
    mesh=_MESH,
    compiler_params=_SC_PARAMS,
    scratch_types=[
        pltpu.VMEM((KB, B), jnp.int32),
        pltpu.VMEM((B, 16), jnp.float32),
        pltpu.VMEM((B, 16), jnp.float32),
        pltpu.VMEM_SHARED((NA, 16), jnp.float32),
    ],
)(_deg_body)


def _make_scatter(C):
    def body(*refs):
        src_hbm, dst_hbm, z_hbm = refs[0], refs[1], refs[2]
        tabs = refs[3:3 + C]
        outs = refs[3 + C:3 + 2 * C]
        src_v, dst_v, zeros_v, buf, acc = refs[3 + 2 * C:]
        cc = lax.axis_index("c")
        sid = lax.axis_index("s")
        wid = cc * NS + sid
        pltpu.sync_copy(src_hbm.at[wid], src_v)
        pltpu.sync_copy(dst_hbm.at[wid], dst_v)
        pltpu.sync_copy(z_hbm, zeros_v)
        for c in range(C):
            _zero_acc(zeros_v, acc, sid)
            plsc.subcore_barrier()
            tab = tabs[c]

            def bat(j, car):
                pltpu.sync_copy(tab.at[src_v.at[j]], buf)
                pltpu.sync_copy(buf, acc.at[dst_v.at[j]], add=True)
                return car

            lax.fori_loop(0, KB, bat, 0)
            plsc.subcore_barrier()
            pltpu.sync_copy(acc.at[pl.ds(sid * 3128, 3128)],
                            outs[c].at[cc, pl.ds(sid * 3128, 3128)])
            plsc.subcore_barrier()

    return pl.kernel(
        body,
        out_type=[jax.ShapeDtypeStruct((NC, NA, F), jnp.float32)] * C,
        mesh=_MESH,
        compiler_params=_SC_PARAMS,
        scratch_types=[
            pltpu.VMEM((KB, B), jnp.int32),
            pltpu.VMEM((KB, B), jnp.int32),
            pltpu.VMEM((B, F), jnp.float32),
            pltpu.VMEM((B, F), jnp.float32),
            pltpu.VMEM_SHARED((NA, F), jnp.float32),
        ],
    )


_scatter = {C: _make_scatter(C) for C in (2, 4, 8)}


def _row_spec(shape):
    nd = len(shape)
    if nd == 2:
        return pl.BlockSpec((RB, shape[1]), lambda i: (i, 0))
    return pl.BlockSpec((shape[0], RB, shape[2]), lambda i: (0, i, 0))


def _full_spec(shape):
    return pl.BlockSpec(shape, lambda i: (0,) * len(shape))


C1 = 64 // F


def _k0_body(x_ref, dA_ref, dB_ref, W_ref, dinv_ref, *t_refs):
    deg = dA_ref[...][:, 0:1] + dB_ref[...][:, 0:1] + 1.0
    dinv = lax.rsqrt(deg)
    dinv_ref[...] = dinv
    xw = jnp.dot(x_ref[...], W_ref[...],
                 preferred_element_type=jnp.float32) * dinv
    for c in range(C1):
        t_refs[c][...] = xw[:, F * c:F * (c + 1)]


def _k0(x, degp, W1):
    return pl.pallas_call(
        _k0_body,
        grid=(GRID,),
        in_specs=[_row_spec(x.shape),
                  pl.BlockSpec((RB, 16), lambda i: (i, 0)),
                  pl.BlockSpec((RB, 16), lambda i: (i, 0)),
                  _full_spec(W1.shape)],
        out_specs=[_row_spec((N, 1))] + [_row_spec((NA, F))] * C1,
        out_shape=[jax.ShapeDtypeStruct((N, 1), jnp.float32)]
        + [jax.ShapeDtypeStruct((NA, F), jnp.float32)] * C1,
    )(x, degp[0], degp[1], W1)


def _pre_act(dinv_ref, b_ref, p_refs, t_refs):
    parts = [p[...][0] + p[...][1] + t[...] for p, t in zip(p_refs, t_refs)]
    t = parts[0] if len(parts) == 1 else jnp.concatenate(parts, axis=1)
    return t * dinv_ref[...] + b_ref[...]


def _make_stats(C):
    do = F * C

    def body(*refs):
        dinv_ref, b_ref = refs[0], refs[1]
        p_refs = refs[2:2 + C]
        t_refs = refs[2 + C:2 + 2 * C]
        stats_ref, acc_ref = refs[2 + 2 * C], refs[3 + 2 * C]
        i = pl.program_id(0)
        pre = _pre_act(dinv_ref, b_ref, p_refs, t_refs)

        @pl.when(i == 0)
        def _():
            acc_ref[...] = jnp.zeros_like(acc_ref)

        acc_ref[...] += jnp.stack(
            [jnp.sum(pre, axis=0), jnp.sum(pre * pre, axis=0)])

        @pl.when(i == GRID - 1)
        def _():
            stats_ref[...] = acc_ref[...]

    def call(dinv, b, parts, tabs):
        return pl.pallas_call(
            body,
            grid=(GRID,),
            in_specs=[_row_spec((N, 1)), _full_spec((1, do))]
            + [_row_spec((NC, NA, F))] * C + [_row_spec((NA, F))] * C,
            out_specs=_full_spec((2, do)),
            out_shape=jax.ShapeDtypeStruct((2, do), jnp.float32),
            scratch_shapes=[pltpu.VMEM((2, do), jnp.float32)],
        )(dinv, b, *parts, *tabs)

    return call


def _bn_apply(pre, stats_ref, g_ref, be_ref):
    m = stats_ref[...][0:1, :] / N
    v = stats_ref[...][1:2, :] / N - m * m
    rstd = lax.rsqrt(v + EPS)
    return (pre - m) * rstd * g_ref[...] + be_ref[...]


def _make_apply(C, C_next):
    do = F * C

    def body(*refs):
        dinv_ref, b_ref, g_ref, be_ref, stats_ref, W_ref = refs[:6]
        p_refs = refs[6:6 + C]
        t_refs = refs[6 + C:6 + 2 * C]
        o_refs = refs[6 + 2 * C:]
        pre = _pre_act(dinv_ref, b_ref, p_refs, t_refs)
        h = jnp.maximum(_bn_apply(pre, stats_ref, g_ref, be_ref), 0.0)
        xw = jnp.dot(h, W_ref[...],
                     preferred_element_type=jnp.float32) * dinv_ref[...]
        for c2 in range(C_next):
            o_refs[c2][...] = xw[:, F * c2:F * (c2 + 1)]

    def call(dinv, b, g, be, stats, W, parts, tabs):
        return pl.pallas_call(
            body,
            grid=(GRID,),
            in_specs=[_row_spec((N, 1)), _full_spec((1, do)),
                      _full_spec((1, do)), _full_spec((1, do)),
                      _full_spec((2, do)), _full_spec(W.shape)]
            + [_row_spec((NC, NA, F))] * C + [_row_spec((NA, F))] * C,
            out_specs=[_row_spec((NA, F))] * C_next,
            out_shape=[jax.ShapeDtypeStruct((NA, F), jnp.float32)] * C_next,
        )(dinv, b, g, be, stats, W, *parts, *tabs)

    return call


C4 = 32 // F


def _apply4_body(*refs):
    dinv_ref, b_ref, g_ref, be_ref, stats_ref, cW_ref, cb_ref = refs[:7]
    p_refs = refs[7:7 + C4]
    t_refs = refs[7 + C4:7 + 2 * C4]
    d1_ref, cstats_ref, acc_ref = refs[7 + 2 * C4:]
    i = pl.program_id(0)
    pre = _pre_act(dinv_ref, b_ref, p_refs, t_refs)
    h4 = jnp.maximum(_bn_apply(pre, stats_ref, g_ref, be_ref), 0.0)
    d1 = jnp.maximum(
        jnp.dot(h4, cW_ref[...], preferred_element_type=jnp.float32)
        + cb_ref[...], 0.0)
    d1_ref[...] = d1

    @pl.when(i == 0)
    def _():
        acc_ref[...] = jnp.zeros_like(acc_ref)

    acc_ref[...] += jnp.stack([jnp.sum(d1, axis=0), jnp.sum(d1 * d1, axis=0)])

    @pl.when(i == GRID - 1)
    def _():
        cstats_ref[...] = acc_ref[...]


def _apply4(dinv, b, g, be, stats, cW1, cb1, parts, tabs):
    return pl.pallas_call(
        _apply4_body,
        grid=(GRID,),
        in_specs=[_row_spec((N, 1)), _full_spec((1, 32)), _full_spec((1, 32)),
                  _full_spec((1, 32)), _full_spec((2, 32)),
                  _full_spec(cW1.shape), _full_spec((1, 32))]
        + [_row_spec((NC, NA, F))] * C4 + [_row_spec((NA, F))] * C4,
        out_specs=[_row_spec((N, 32)), _full_spec((2, 32))],
        out_shape=[jax.ShapeDtypeStruct((N, 32), jnp.float32),
                   jax.ShapeDtypeStruct((2, 32), jnp.float32)],
        scratch_shapes=[pltpu.VMEM((2, 32), jnp.float32)],
    )(dinv, b, g, be, stats, cW1, cb1, *parts, *tabs)


def _mlp_body(d_ref, stats_ref, g_ref, be_ref, W_ref, wb_ref,
              d2_ref, cstats_ref, acc_ref):
    i = pl.program_id(0)
    e = _bn_apply(d_ref[...], stats_ref, g_ref, be_ref)
    d2 = jnp.maximum(
        jnp.dot(e, W_ref[...], preferred_element_type=jnp.float32)
        + wb_ref[...], 0.0)
    d2_ref[...] = d2

    @pl.when(i == 0)
    def _():
        acc_ref[...] = jnp.zeros_like(acc_ref)

    acc_ref[...] += jnp.stack([jnp.sum(d2, axis=0), jnp.sum(d2 * d2, axis=0)])

    @pl.when(i == GRID - 1)
    def _():
        cstats_ref[...] = acc_ref[...]


def _mlp(d, stats, g, be, W, wb, dn):
    di = d.shape[1]
    return pl.pallas_call(
        _mlp_body,
        grid=(GRID,),
        in_specs=[_row_spec((N, di)), _full_spec((2, di)),
                  _full_spec((1, di)), _full_spec((1, di)),
                  _full_spec(W.shape), _full_spec((1, dn))],
        out_specs=[_row_spec((N, dn)), _full_spec((2, dn))],
        out_shape=[jax.ShapeDtypeStruct((N, dn), jnp.float32),
                   jax.ShapeDtypeStruct((2, dn), jnp.float32)],
        scratch_shapes=[pltpu.VMEM((2, dn), jnp.float32)],
    )(d, stats, g, be, W, wb)


def _final_body(d_ref, stats_ref, g_ref, be_ref, W_ref, wb_ref, out_ref):
    e = _bn_apply(d_ref[...], stats_ref, g_ref, be_ref)
    out_ref[...] = (jnp.dot(e, W_ref[...], preferred_element_type=jnp.float32)
                    + wb_ref[...])


def _final(d, stats, g, be, W, wb):
    di, dn = W.shape
    return pl.pallas_call(
        _final_body,
        grid=(GRID,),
        in_specs=[_row_spec((N, di)), _full_spec((2, di)),
                  _full_spec((1, di)), _full_spec((1, di)),
                  _full_spec(W.shape), _full_spec((1, dn))],
        out_specs=_row_spec((N, dn)),
        out_shape=jax.ShapeDtypeStruct((N, dn), jnp.float32),
    )(d, stats, g, be, W, wb)


def kernel(x, edge_index, W1, b1, g1, be1, W2, b2, g2, be2, W3, b3, g3, be3,
           W4, b4, g4, be4, cW1, cb1, cW2, cb2, cW3, cb3, cg1, cbe1,
           cg2, cbe2):
    pad = jnp.full((EPAD - E,), N, jnp.int32)
    src_t = jnp.concatenate([edge_index[0], pad]).reshape(NW, KB, B)
    dst_t = jnp.concatenate([edge_index[1], pad]).reshape(NW, KB, B)
    ones16 = jnp.ones((B, 16), jnp.float32)
    z16 = jnp.zeros((B, 16), jnp.float32)
    zF = jnp.zeros((B, F), jnp.float32)
    r2 = lambda v: v.reshape(1, -1)

    degp = _deg_kernel(dst_t, ones16, z16)
    k0_out = _k0(x, degp, W1)
    dinv, tabs = k0_out[0], list(k0_out[1:])

    layer_params = [(b1, g1, be1, 4, W2, 8), (b2, g2, be2, 8, W3, 4),
                    (b3, g3, be3, 4, W4, 2)]
    for b_, g_, be_, C, Wn, Cn in layer_params:
        parts = _scatter[C](src_t, dst_t, zF, *tabs)
        parts = list(parts) if isinstance(parts, (list, tuple)) else [parts]
        stats = _make_stats(C)(dinv, r2(b_), parts, tabs)
        tabs = _make_apply(C, Cn)(dinv, r2(b_), r2(g_), r2(be_), stats, Wn,
                                  parts, tabs)
        tabs = list(tabs) if isinstance(tabs, (list, tuple)) else [tabs]

    parts = _scatter[2](src_t, dst_t, zF, *tabs)
    parts = list(parts) if isinstance(parts, (list, tuple)) else [parts]
    stats4 = _make_stats(2)(dinv, r2(b4), parts, tabs)
    d1, cs1 = _apply4(dinv, r2(b4), r2(g4), r2(be4), stats4, cW1, r2(cb1),
                      parts, tabs)
    d2, cs2 = _mlp(d1, cs1, r2(cg1), r2(cbe1), cW2, r2(cb2), 16)
    out = _final(d2, cs2, r2(cg2), r2(cbe2), cW3, r2(cb3))
    return out
